# baseline (speedup 1.0000x reference)
import jax
import jax.numpy as jnp
import numpy as np
from jax.experimental import pallas as pl
from jax.experimental.pallas import tpu as pltpu

COMPRESSION_BS = 32
SLIDING_STRIDE = 16
SELECTED_BSIZE = 64
SELECTED_COUNT = 16
SLIDING_WINDOW = 512
S, DIM, HEADS, DH = 2048, 768, 12, 64
N_CMP = (S - COMPRESSION_BS) // SLIDING_STRIDE + 2
N_SLC = S // SELECTED_BSIZE
BQ = 256
BK = 256
NEG = float(np.finfo(np.float32).min)
SCALE = 1.0 / np.sqrt(DH)


HI = jax.lax.Precision.HIGHEST


def _dot_t(a, b, prec=None):
    return jax.lax.dot_general(a, b, (((1,), (1,)), ((), ())),
                               precision=prec, preferred_element_type=jnp.float32)


def _proj_body(x_ref, w_ref, out_ref):
    out_ref[0] = _dot_t(x_ref[...], w_ref[0])


def _proj(x, wqkv3):
    nb = S // BQ
    return pl.pallas_call(
        _proj_body,
        grid=(3 * HEADS, nb),
        in_specs=[
            pl.BlockSpec((BQ, DIM), lambda c, r: (r, 0)),
            pl.BlockSpec((1, DH, DIM), lambda c, r: (c, 0, 0)),
        ],
        out_specs=pl.BlockSpec((1, BQ, DH), lambda c, r: (c, r, 0)),
        out_shape=jax.ShapeDtypeStruct((3 * HEADS, S, DH), jnp.float32),
    )(x, wqkv3)


def _gate_body(x_ref, wg1_ref, wg2_ref, g_ref):
    h1 = jax.nn.silu(_dot_t(x_ref[...], wg1_ref[...]))
    g_ref[...] = jax.nn.sigmoid(_dot_t(h1, wg2_ref[...]))


def _gate(x, wg1, wg2):
    nb = S // BQ
    return pl.pallas_call(
        _gate_body,
        grid=(nb,),
        in_specs=[
            pl.BlockSpec((BQ, DIM), lambda r: (r, 0)),
            pl.BlockSpec((DIM // 2, DIM), lambda r: (0, 0)),
            pl.BlockSpec((3 * HEADS, DIM // 2), lambda r: (0, 0)),
        ],
        out_specs=pl.BlockSpec((BQ, 3 * HEADS), lambda r: (r, 0)),
        out_shape=jax.ShapeDtypeStruct((S, 3 * HEADS), jnp.float32),
    )(x, wg1, wg2)


def _compress_body(kv_ref, pe_ref, down_ref, stop_ref, out_ref):
    t = kv_ref[0]
    c = t.reshape(S // SLIDING_STRIDE, SLIDING_STRIDE, DH)
    n = (S - COMPRESSION_BS) // SLIDING_STRIDE + 1
    w = jnp.concatenate([c[:n], c[1:n + 1]], axis=1)
    w = w + pe_ref[0][None]
    w = jnp.concatenate([jnp.zeros((1, COMPRESSION_BS, DH), jnp.float32), w],
                        axis=0)
    x = w.reshape(N_CMP * COMPRESSION_BS, DH)
    for i in range(5):
        y = x.reshape(x.shape[0] // 2, 2, DH)
        a = y[:, 0, :]
        b = y[:, 1, :]
        x = jax.nn.silu(_dot_t(a, down_ref[0, i, :, :DH])
                        + _dot_t(b, down_ref[0, i, :, DH:]))
    out_ref[0, 0] = _dot_t(x, stop_ref[0])


def _compress(qkv3, pe, down, stop):
    return pl.pallas_call(
        _compress_body,
        grid=(2, HEADS),
        in_specs=[
            pl.BlockSpec((1, S, DH), lambda t, h: (HEADS * (t + 1) + h, 0, 0)),
            pl.BlockSpec((1, COMPRESSION_BS, DH), lambda t, h: (t, 0, 0)),
            pl.BlockSpec((1, 5, DH, 2 * DH), lambda t, h: (t, 0, 0, 0)),
            pl.BlockSpec((1, DH, DH), lambda t, h: (t, 0, 0)),
        ],
        out_specs=pl.BlockSpec((1, 1, N_CMP, DH), lambda t, h: (t, h, 0, 0)),
        out_shape=jax.ShapeDtypeStruct((2, HEADS, N_CMP, DH), jnp.float32),
    )(qkv3, pe, down, stop)


def _masked_softmax(s, mask):
    sm = jnp.where(mask, s, NEG)
    m = jnp.max(sm, axis=-1, keepdims=True)
    e = jnp.where(mask, jnp.exp(s - m), 0.0)
    return e / jnp.maximum(jnp.sum(e, axis=-1, keepdims=True), 1e-30)


def _attn_body(q_ref, k_ref, v_ref, kc_ref, vc_ref, g_ref, out_ref):
    qb = pl.program_id(1)
    q = q_ref[0]
    qg = qb * BQ + jax.lax.broadcasted_iota(jnp.int32, (BQ, 1), 0)

    kc = kc_ref[0, 0]
    vc = vc_ref[0, 0]
    s_cmp = _dot_t(q, kc) * SCALE
    jc = jax.lax.broadcasted_iota(jnp.int32, (BQ, N_CMP), 1)
    cmask = (jc < qg // SLIDING_STRIDE) | ((jc == 0) & (qg < SLIDING_WINDOW))
    p_cmp = _masked_softmax(s_cmp, cmask)
    o_cmp = jnp.dot(p_cmp, vc, preferred_element_type=jnp.float32)

    tt = (jax.lax.broadcasted_iota(jnp.int32, (N_CMP, N_SLC), 0) + 1
          - 4 * jax.lax.broadcasted_iota(jnp.int32, (N_CMP, N_SLC), 1))
    conv = jnp.where((tt >= 0) & (tt <= 4),
                     jnp.where((tt >= 1) & (tt <= 3), 2.0, 1.0), 0.0)
    p_slc = jnp.dot(p_cmp, conv, preferred_element_type=jnp.float32, precision=HI)

    pj = p_slc[:, :, None]
    plc = p_slc[:, None, :]
    li = jax.lax.broadcasted_iota(jnp.int32, (BQ, N_SLC, N_SLC), 2)
    ji = jax.lax.broadcasted_iota(jnp.int32, (BQ, N_SLC, N_SLC), 1)
    beats = (plc > pj) | ((plc == pj) & (li < ji))
    cnt = jnp.sum(beats.astype(jnp.float32), axis=2)
    jb = jax.lax.broadcasted_iota(jnp.int32, (BQ, N_SLC), 1)
    sel = ((cnt < SELECTED_COUNT) & (jb < qg // SELECTED_BSIZE)).astype(jnp.float32)

    def tile(kb, carry):
        ms, ls, accs, mw, lw, accw = carry
        k_t = k_ref[0, pl.ds(kb * BK, BK), :]
        v_t = v_ref[0, pl.ds(kb * BK, BK), :]
        s = _dot_t(q, k_t) * SCALE
        kg = kb * BK + jax.lax.broadcasted_iota(jnp.int32, (BQ, BK), 1)

        def upd(m, l, acc, mask):
            sm = jnp.where(mask, s, NEG)
            mt = jnp.max(sm, axis=-1, keepdims=True)
            mn = jnp.maximum(m, mt)
            p = jnp.where(mask, jnp.exp(s - mn), 0.0)
            alpha = jnp.exp(m - mn)
            l2 = l * alpha + jnp.sum(p, axis=-1, keepdims=True)
            acc2 = acc * alpha + jnp.dot(p, v_t, preferred_element_type=jnp.float32)
            return mn, l2, acc2

        ci = (jax.lax.broadcasted_iota(jnp.int32, (N_SLC, BK), 1)
              // SELECTED_BSIZE + kb * (BK // SELECTED_BSIZE))
        jt = jax.lax.broadcasted_iota(jnp.int32, (N_SLC, BK), 0)
        et = (ci == jt).astype(jnp.float32)
        mask_s = jnp.dot(sel, et, preferred_element_type=jnp.float32) > 0.5
        ms, ls, accs = upd(ms, ls, accs, mask_s)

        def do_swa(args):
            mw, lw, accw = args
            mask_w = (qg >= kg) & (qg - kg <= SLIDING_WINDOW)
            return upd(mw, lw, accw, mask_w)

        mw, lw, accw = jax.lax.cond(kb >= qb - (SLIDING_WINDOW // BK),
                                    do_swa, lambda a: a, (mw, lw, accw))
        return ms, ls, accs, mw, lw, accw

    z1 = jnp.full((BQ, 1), NEG)
    z0 = jnp.zeros((BQ, 1), jnp.float32)
    za = jnp.zeros((BQ, DH), jnp.float32)
    carry = jax.lax.fori_loop(0, qb + 1, tile, (z1, z0, za, z1, z0, za))
    ms, ls, accs, mw, lw, accw = carry
    o_slc = accs / jnp.maximum(ls, 1e-30)
    o_swa = accw / jnp.maximum(lw, 1e-30)

    g = g_ref[0]
    out_ref[0] = (g[:, 0:1] * o_cmp + g[:, 1:2] * o_slc + g[:, 2:3] * o_swa)


def _attention(qkv3, kvcmp, g3):
    nq = S // BQ
    return pl.pallas_call(
        _attn_body,
        grid=(HEADS, nq),
        in_specs=[
            pl.BlockSpec((1, BQ, DH), lambda h, r: (h, r, 0)),
            pl.BlockSpec((1, S, DH), lambda h, r: (HEADS + h, 0, 0)),
            pl.BlockSpec((1, S, DH), lambda h, r: (2 * HEADS + h, 0, 0)),
            pl.BlockSpec((1, 1, N_CMP, DH), lambda h, r: (0, h, 0, 0)),
            pl.BlockSpec((1, 1, N_CMP, DH), lambda h, r: (1, h, 0, 0)),
            pl.BlockSpec((1, BQ, 3), lambda h, r: (h, r, 0)),
        ],
        out_specs=pl.BlockSpec((1, BQ, DH), lambda h, r: (h, r, 0)),
        out_shape=jax.ShapeDtypeStruct((HEADS, S, DH), jnp.float32),
    )(qkv3, qkv3, qkv3, kvcmp, kvcmp, g3)


def _oproj_body(r_ref, wo_ref, out_ref):
    acc = jnp.zeros((BQ, DIM), jnp.float32)
    for h in range(HEADS):
        acc = acc + jnp.dot(r_ref[h], wo_ref[h],
                            preferred_element_type=jnp.float32)
    out_ref[...] = acc


def _oproj(res3, wo3):
    nb = S // BQ
    return pl.pallas_call(
        _oproj_body,
        grid=(nb,),
        in_specs=[
            pl.BlockSpec((HEADS, BQ, DH), lambda r: (0, r, 0)),
            pl.BlockSpec((HEADS, DH, DIM), lambda r: (0, 0, 0)),
        ],
        out_specs=pl.BlockSpec((BQ, DIM), lambda r: (r, 0)),
        out_shape=jax.ShapeDtypeStruct((S, DIM), jnp.float32),
    )(res3, wo3)


@jax.jit
def kernel(x, Wq, Wk, Wv, Wo, Wg1, Wg2, pe_k, down_k, stop_k, pe_v, down_v, stop_v):
    x2 = x[0]
    wqkv3 = jnp.concatenate([Wq, Wk, Wv], axis=0).reshape(3 * HEADS, DH, DIM)
    qkv3 = _proj(x2, wqkv3)
    g = _gate(x2, Wg1, Wg2)
    pe = jnp.stack([pe_k, pe_v])
    down = jnp.stack([down_k, down_v])
    stop = jnp.stack([stop_k, stop_v])
    kvcmp = _compress(qkv3, pe, down, stop)
    g3 = g.reshape(S, HEADS, 3).transpose(1, 0, 2)
    res3 = _attention(qkv3, kvcmp, g3)
    wo3 = Wo.reshape(DIM, HEADS, DH).transpose(1, 2, 0)
    out = _oproj(res3, wo3)
    return out[None]


# speedup vs baseline: 1.6935x; 1.6935x over previous
import jax
import jax.numpy as jnp
import numpy as np
from jax.experimental import pallas as pl
from jax.experimental.pallas import tpu as pltpu

COMPRESSION_BS = 32
SLIDING_STRIDE = 16
SELECTED_BSIZE = 64
SELECTED_COUNT = 16
SLIDING_WINDOW = 512
S, DIM, HEADS, DH = 2048, 768, 12, 64
N_CMP = (S - COMPRESSION_BS) // SLIDING_STRIDE + 2
N_SLC = S // SELECTED_BSIZE
BQ = 256
BK = 512
NEG = float(np.finfo(np.float32).min)
SCALE = 1.0 / np.sqrt(DH)
HI = jax.lax.Precision.HIGHEST


def _dot_t(a, b, prec=None):
    return jax.lax.dot_general(a, b, (((1,), (1,)), ((), ())),
                               precision=prec, preferred_element_type=jnp.float32)


def _iota(shape, dim):
    return jax.lax.broadcasted_iota(jnp.int32, shape, dim)


def _proj_body(x_ref, w_ref, wg1_ref, wg2_ref, qkv_ref, g_ref):
    x = x_ref[...]
    for c in range(3 * HEADS):
        qkv_ref[c] = _dot_t(x, w_ref[c])
    h1 = jax.nn.silu(_dot_t(x, wg1_ref[...]))
    g_ref[...] = jax.nn.sigmoid(_dot_t(h1, wg2_ref[...]))


def _proj(x, wqkv3, wg1, wg2):
    nb = S // BQ
    return pl.pallas_call(
        _proj_body,
        grid=(nb,),
        in_specs=[
            pl.BlockSpec((BQ, DIM), lambda r: (r, 0)),
            pl.BlockSpec((3 * HEADS, DH, DIM), lambda r: (0, 0, 0)),
            pl.BlockSpec((DIM // 2, DIM), lambda r: (0, 0)),
            pl.BlockSpec((3 * HEADS, DIM // 2), lambda r: (0, 0)),
        ],
        out_specs=[
            pl.BlockSpec((3 * HEADS, BQ, DH), lambda r: (0, r, 0)),
            pl.BlockSpec((BQ, 3 * HEADS), lambda r: (r, 0)),
        ],
        out_shape=[
            jax.ShapeDtypeStruct((3 * HEADS, S, DH), jnp.float32),
            jax.ShapeDtypeStruct((S, 3 * HEADS), jnp.float32),
        ],
    )(x, wqkv3, wg1, wg2)


def _compress_body(kv_ref, pe_ref, down_ref, stop_ref, out_ref):
    t = kv_ref[0]
    c = t.reshape(S // SLIDING_STRIDE, SLIDING_STRIDE, DH)
    n = (S - COMPRESSION_BS) // SLIDING_STRIDE + 1
    w = jnp.concatenate([c[:n], c[1:n + 1]], axis=1)
    w = w + pe_ref[0][None]
    w = jnp.concatenate([jnp.zeros((1, COMPRESSION_BS, DH), jnp.float32), w],
                        axis=0)
    x = w.reshape(N_CMP * COMPRESSION_BS, DH)
    for i in range(5):
        y = x.reshape(x.shape[0] // 2, 2, DH)
        a = y[:, 0, :]
        b = y[:, 1, :]
        x = jax.nn.silu(_dot_t(a, down_ref[0, i, :, :DH])
                        + _dot_t(b, down_ref[0, i, :, DH:]))
    out_ref[0, 0] = _dot_t(x, stop_ref[0])


def _compress(qkv3, pe, down, stop):
    return pl.pallas_call(
        _compress_body,
        grid=(2, HEADS),
        in_specs=[
            pl.BlockSpec((1, S, DH), lambda t, h: (HEADS * (t + 1) + h, 0, 0)),
            pl.BlockSpec((1, COMPRESSION_BS, DH), lambda t, h: (t, 0, 0)),
            pl.BlockSpec((1, 5, DH, 2 * DH), lambda t, h: (t, 0, 0, 0)),
            pl.BlockSpec((1, DH, DH), lambda t, h: (t, 0, 0)),
        ],
        out_specs=pl.BlockSpec((1, 1, N_CMP, DH), lambda t, h: (t, h, 0, 0)),
        out_shape=jax.ShapeDtypeStruct((2, HEADS, N_CMP, DH), jnp.float32),
    )(qkv3, pe, down, stop)


def _masked_softmax(s, mask):
    sm = jnp.where(mask, s, NEG)
    m = jnp.max(sm, axis=-1, keepdims=True)
    e = jnp.where(mask, jnp.exp(s - m), 0.0)
    return e / jnp.maximum(jnp.sum(e, axis=-1, keepdims=True), 1e-30)


def _select_body(q_ref, kc_ref, vc_ref, oc_ref, sel_ref):
    q = q_ref[0]
    kc = kc_ref[0, 0]
    vc = vc_ref[0, 0]
    s_cmp = _dot_t(q, kc) * SCALE
    qg = _iota((S, 1), 0)
    jc = _iota((S, N_CMP), 1)
    cmask = (jc < qg // SLIDING_STRIDE) | ((jc == 0) & (qg < SLIDING_WINDOW))
    p_cmp = _masked_softmax(s_cmp, cmask)
    oc_ref[0] = jnp.dot(p_cmp, vc, preferred_element_type=jnp.float32)

    tt = _iota((N_CMP, N_SLC), 0) + 1 - 4 * _iota((N_CMP, N_SLC), 1)
    conv = jnp.where((tt >= 0) & (tt <= 4),
                     jnp.where((tt >= 1) & (tt <= 3), 2.0, 1.0), 0.0)
    p_slc = jnp.dot(p_cmp, conv, preferred_element_type=jnp.float32,
                    precision=HI)

    pj = p_slc[:, :, None]
    plc = p_slc[:, None, :]
    li = _iota((S, N_SLC, N_SLC), 2)
    ji = _iota((S, N_SLC, N_SLC), 1)
    beats = (plc > pj) | ((plc == pj) & (li < ji))
    cnt = jnp.sum(beats.astype(jnp.float32), axis=2)
    jb = _iota((S, N_SLC), 1)
    sel_ref[0] = ((cnt < SELECTED_COUNT)
                  & (jb < qg // SELECTED_BSIZE)).astype(jnp.float32)


def _select(qkv3, kvcmp):
    return pl.pallas_call(
        _select_body,
        grid=(HEADS,),
        in_specs=[
            pl.BlockSpec((1, S, DH), lambda h: (h, 0, 0)),
            pl.BlockSpec((1, 1, N_CMP, DH), lambda h: (0, h, 0, 0)),
            pl.BlockSpec((1, 1, N_CMP, DH), lambda h: (1, h, 0, 0)),
        ],
        out_specs=[
            pl.BlockSpec((1, S, DH), lambda h: (h, 0, 0)),
            pl.BlockSpec((1, S, N_SLC), lambda h: (h, 0, 0)),
        ],
        out_shape=[
            jax.ShapeDtypeStruct((HEADS, S, DH), jnp.float32),
            jax.ShapeDtypeStruct((HEADS, S, N_SLC), jnp.float32),
        ],
    )(qkv3, kvcmp, kvcmp)


def _flash_body(q_ref, k_ref, v_ref, sel_ref, oc_ref, g_ref, out_ref):
    qb = pl.program_id(1)
    q = q_ref[0] * SCALE
    sel = sel_ref[0]
    qg = qb * BQ + _iota((BQ, 1), 0)
    nt = qb // 2 + 1

    def selmask(t):
        ci = _iota((N_SLC, BK), 1) // SELECTED_BSIZE + t * (BK // SELECTED_BSIZE)
        jt = _iota((N_SLC, BK), 0)
        return jnp.dot(sel, (ci == jt).astype(jnp.float32),
                       preferred_element_type=jnp.float32)

    def make_tile(diag, swa, winmask):
        def f(t, carry):
            ms, ls, accs, mw, lw, accw = carry
            k_t = k_ref[0, pl.ds(t * BK, BK), :]
            v_t = v_ref[0, pl.ds(t * BK, BK), :]
            s = _dot_t(q, k_t)
            kg = t * BK + _iota((BQ, BK), 1)
            if diag:
                cm = qg >= kg
                mt = jnp.max(jnp.where(cm, s, NEG), axis=-1, keepdims=True)
            else:
                mt = jnp.max(s, axis=-1, keepdims=True)
            mn = jnp.maximum(ms, mt)
            alpha = jnp.exp(ms - mn)
            if diag:
                e = jnp.where(cm, jnp.exp(s - mn), 0.0)
            else:
                e = jnp.exp(s - mn)
            es = e * selmask(t)
            ls = ls * alpha + jnp.sum(es, axis=-1, keepdims=True)
            accs = accs * alpha + jnp.dot(es, v_t,
                                          preferred_element_type=jnp.float32)
            if swa:
                aw = jnp.exp(mw - mn)
                if winmask:
                    ew = e * (qg - kg <= SLIDING_WINDOW).astype(jnp.float32)
                else:
                    ew = e
                lw = lw * aw + jnp.sum(ew, axis=-1, keepdims=True)
                accw = accw * aw + jnp.dot(ew, v_t,
                                           preferred_element_type=jnp.float32)
                mw = mn
            return mn, ls, accs, mw, lw, accw
        return f

    z1 = jnp.full((BQ, 1), NEG)
    z0 = jnp.zeros((BQ, 1), jnp.float32)
    za = jnp.zeros((BQ, DH), jnp.float32)
    carry = (z1, z0, za, z1, z0, za)
    carry = jax.lax.fori_loop(0, nt - 2, make_tile(False, False, False), carry)
    carry = jax.lax.cond(nt >= 2,
                         lambda c: make_tile(False, True, True)(nt - 2, c),
                         lambda c: c, carry)
    ms, ls, accs, mw, lw, accw = make_tile(True, True, False)(nt - 1, carry)

    o_slc = accs / jnp.maximum(ls, 1e-30)
    o_swa = accw / jnp.maximum(lw, 1e-30)
    g = g_ref[0]
    out_ref[0] = (g[:, 0:1] * oc_ref[0] + g[:, 1:2] * o_slc
                  + g[:, 2:3] * o_swa)


def _flash(qkv3, selm, ocmp, g3):
    nq = S // BQ
    return pl.pallas_call(
        _flash_body,
        grid=(HEADS, nq),
        in_specs=[
            pl.BlockSpec((1, BQ, DH), lambda h, r: (h, r, 0)),
            pl.BlockSpec((1, S, DH), lambda h, r: (HEADS + h, 0, 0)),
            pl.BlockSpec((1, S, DH), lambda h, r: (2 * HEADS + h, 0, 0)),
            pl.BlockSpec((1, BQ, N_SLC), lambda h, r: (h, r, 0)),
            pl.BlockSpec((1, BQ, DH), lambda h, r: (h, r, 0)),
            pl.BlockSpec((1, BQ, 3), lambda h, r: (h, r, 0)),
        ],
        out_specs=pl.BlockSpec((1, BQ, DH), lambda h, r: (h, r, 0)),
        out_shape=jax.ShapeDtypeStruct((HEADS, S, DH), jnp.float32),
    )(qkv3, qkv3, qkv3, selm, ocmp, g3)


def _oproj_body(r_ref, wo_ref, out_ref):
    acc = jnp.zeros((BQ, DIM), jnp.float32)
    for h in range(HEADS):
        acc = acc + jnp.dot(r_ref[h], wo_ref[h],
                            preferred_element_type=jnp.float32)
    out_ref[...] = acc


def _oproj(res3, wo3):
    nb = S // BQ
    return pl.pallas_call(
        _oproj_body,
        grid=(nb,),
        in_specs=[
            pl.BlockSpec((HEADS, BQ, DH), lambda r: (0, r, 0)),
            pl.BlockSpec((HEADS, DH, DIM), lambda r: (0, 0, 0)),
        ],
        out_specs=pl.BlockSpec((BQ, DIM), lambda r: (r, 0)),
        out_shape=jax.ShapeDtypeStruct((S, DIM), jnp.float32),
    )(res3, wo3)


@jax.jit
def kernel(x, Wq, Wk, Wv, Wo, Wg1, Wg2, pe_k, down_k, stop_k, pe_v, down_v, stop_v):
    x2 = x[0]
    wqkv3 = jnp.concatenate([Wq, Wk, Wv], axis=0).reshape(3 * HEADS, DH, DIM)
    qkv3, g = _proj(x2, wqkv3, Wg1, Wg2)
    pe = jnp.stack([pe_k, pe_v])
    down = jnp.stack([down_k, down_v])
    stop = jnp.stack([stop_k, stop_v])
    kvcmp = _compress(qkv3, pe, down, stop)
    ocmp, selm = _select(qkv3, kvcmp)
    g3 = g.reshape(S, HEADS, 3).transpose(1, 0, 2)
    res3 = _flash(qkv3, selm, ocmp, g3)
    wo3 = Wo.reshape(DIM, HEADS, DH).transpose(1, 2, 0)
    out = _oproj(res3, wo3)
    return out[None]
